# baseline (speedup 1.0000x reference)
import functools

import jax
import jax.numpy as jnp
from jax import lax
from jax.experimental import pallas as pl
from jax.experimental.pallas import tpu as pltpu
from jax.experimental.pallas import tpu_sc as plsc

NW = 32
LANES = 16


def _ceil_to(x, m):
    return (x + m - 1) // m * m


def _make_deg_kernel(n_nodes, dtbl, e):
    epw = e // NW
    mesh = plsc.VectorSubcoreMesh(core_axis_name="c", subcore_axis_name="s")

    @functools.partial(
        pl.kernel,
        out_type=jax.ShapeDtypeStruct((NW, dtbl), jnp.float32),
        mesh=mesh,
        compiler_params=pltpu.CompilerParams(needs_layout_passes=False),
        scratch_types=[
            pltpu.VMEM((epw,), jnp.int32),
            pltpu.VMEM((dtbl,), jnp.float32),
            pltpu.SemaphoreType.DMA,
        ],
    )
    def deg_kernel(edge_hbm, out_hbm, dst_v, cnt_v, sem):
        cid = lax.axis_index("c")
        sid = lax.axis_index("s")
        wid = sid * 2 + cid
        base = wid * epw
        cp = pltpu.make_async_copy(edge_hbm.at[pl.ds(e + base, epw)], dst_v,
                                   sem)
        cp.start()

        zeros = jnp.zeros((LANES,), jnp.float32)

        def zbody(i, _):
            cnt_v[pl.ds(i * LANES, LANES)] = zeros
            return 0

        lax.fori_loop(0, dtbl // LANES, zbody, 0, unroll=4)
        cp.wait()

        ones = jnp.ones((LANES,), jnp.float32)

        def ebody(i, _):
            d16 = dst_v[pl.ds(i * LANES, LANES)]
            plsc.addupdate_scatter(cnt_v, [d16], ones)
            return 0

        lax.fori_loop(0, epw // LANES, ebody, 0)
        pltpu.sync_copy(cnt_v, out_hbm.at[wid])

    return deg_kernel


def _make_agg_kernel(n_nodes, d, dtbl, e):
    epw = e // NW
    d2 = d // 2
    unroll = 5 if (epw // LANES) % 5 == 0 else 1
    mesh = plsc.VectorSubcoreMesh(core_axis_name="c", subcore_axis_name="s")

    scratch = [pltpu.VMEM((d2, n_nodes), jnp.int32),
               pltpu.VMEM((d, dtbl), jnp.float32),
               pltpu.VMEM((epw,), jnp.int32),
               pltpu.VMEM((epw,), jnp.int32)] + [pltpu.SemaphoreType.DMA] * 3

    @functools.partial(
        pl.kernel,
        out_type=jax.ShapeDtypeStruct((NW, d, dtbl), jnp.float32),
        mesh=mesh,
        compiler_params=pltpu.CompilerParams(needs_layout_passes=False),
        scratch_types=scratch,
    )
    def agg_kernel(gp_hbm, edge_hbm, out_hbm,
                   g_v, acc_v, src_v, dst_v, s0, s2, s3):
        cid = lax.axis_index("c")
        sid = lax.axis_index("s")
        wid = sid * 2 + cid
        base = wid * epw
        copies = [
            pltpu.make_async_copy(gp_hbm, g_v, s0),
            pltpu.make_async_copy(edge_hbm.at[pl.ds(base, epw)], src_v, s2),
            pltpu.make_async_copy(edge_hbm.at[pl.ds(e + base, epw)], dst_v,
                                  s3),
        ]
        for cp in copies:
            cp.start()

        zeros = jnp.zeros((LANES,), jnp.float32)

        def zbody(i, _):
            for j in range(d):
                acc_v[j, pl.ds(i * LANES, LANES)] = zeros
            return 0

        lax.fori_loop(0, dtbl // LANES, zbody, 0, unroll=2)
        for cp in copies:
            cp.wait()

        rows = [jnp.full((LANES,), k, jnp.int32) for k in range(d2)]
        arows = [jnp.full((LANES,), j, jnp.int32) for j in range(d)]
        himask = jnp.full((LANES,), -65536, jnp.int32)

        def ebody(i, _):
            for u in range(unroll):
                off = (i * unroll + u) * LANES
                s16 = src_v[pl.ds(off, LANES)]
                d16 = dst_v[pl.ds(off, LANES)]
                for k in range(d2):
                    w16 = plsc.load_gather(g_v, [rows[k], s16])
                    lo = plsc.bitcast(w16 << 16, jnp.float32)
                    hi = plsc.bitcast(w16 & himask, jnp.float32)
                    plsc.addupdate_scatter(acc_v, [arows[2 * k], d16], lo)
                    plsc.addupdate_scatter(acc_v, [arows[2 * k + 1], d16],
                                           hi)
            return 0

        lax.fori_loop(0, epw // LANES // unroll, ebody, 0)
        pltpu.sync_copy(acc_v, out_hbm.at[wid])

    return agg_kernel


def _mm_body(x_ref, w1_ref, p1_ref):
    p1_ref[...] = lax.dot_general(
        w1_ref[...], x_ref[...],
        dimension_numbers=(((0,), (1,)), ((), ())),
        preferred_element_type=jnp.float32)


def _write_packed(g, gp_ref):
    u = lax.bitcast_convert_type(g, jnp.int32) + 0x8000
    for k in range(g.shape[0] // 2):
        gp_ref[k, :] = (lax.shift_right_logical(u[2 * k], 16)
                        | (u[2 * k + 1] & (-65536)))


def _dis_body(degp_ref, p1_ref, dis_ref, g1_ref, g1p_ref, *, n_nodes):
    deg = jnp.sum(degp_ref[...], axis=0) + 1.0
    dis = lax.rsqrt(deg)
    dis_ref[...] = dis
    g1 = p1_ref[...] * dis[:n_nodes]
    g1_ref[...] = g1
    _write_packed(g1, g1p_ref)


def _post_body(part_ref, g_ref, dis_ref, b_ref, w_ref, o_ref, op_ref,
               *, n_nodes):
    dis = dis_ref[...][:n_nodes]
    s = jnp.sum(part_ref[...], axis=0)[:, :n_nodes] + g_ref[...]
    h = jnp.tanh(dis * s + b_ref[...])
    g_next = dis * lax.dot_general(
        w_ref[...], h,
        dimension_numbers=(((0,), (0,)), ((), ())),
        preferred_element_type=jnp.float32)
    o_ref[...] = g_next
    _write_packed(g_next, op_ref)


def _final_body(part_ref, g_ref, dis_ref, b_ref, wc_ref, bc_ref,
                out_ref, h_ref, *, n_nodes):
    dis = dis_ref[...][:n_nodes]
    s = jnp.sum(part_ref[...], axis=0)[:, :n_nodes] + g_ref[...]
    ht = jnp.tanh(dis * s + b_ref[...])
    h = ht.T
    h_ref[...] = h
    out_ref[...] = jnp.dot(h, wc_ref[...],
                           preferred_element_type=jnp.float32) + bc_ref[...]


def kernel(x, edge_index, W1, b1, W2, b2, W3, b3, Wc, bc):
    n = x.shape[0]
    e = edge_index.shape[1]
    f32 = jnp.float32
    dtbl = _ceil_to(n, LANES)

    edge_flat = edge_index.reshape(-1)
    degp = _make_deg_kernel(n, dtbl, e)(edge_flat)

    p1t = pl.pallas_call(
        _mm_body,
        out_shape=jax.ShapeDtypeStruct((W1.shape[1], n), f32),
    )(x, W1)

    dis, g1, g1p = pl.pallas_call(
        functools.partial(_dis_body, n_nodes=n),
        out_shape=[jax.ShapeDtypeStruct((dtbl,), f32),
                   jax.ShapeDtypeStruct((W1.shape[1], n), f32),
                   jax.ShapeDtypeStruct((W1.shape[1] // 2, n), jnp.int32)],
    )(degp, p1t)

    def layer(gt, gtp, w_next, b):
        d = gt.shape[0]
        parts = _make_agg_kernel(n, d, dtbl, e)(gtp, edge_flat)
        nd = w_next.shape[1]
        return pl.pallas_call(
            functools.partial(_post_body, n_nodes=n),
            out_shape=[jax.ShapeDtypeStruct((nd, n), f32),
                       jax.ShapeDtypeStruct((nd // 2, n), jnp.int32)],
        )(parts, gt, dis, b.reshape(-1, 1), w_next)

    g2, g2p = layer(g1, g1p, W2, b1)
    g3, g3p = layer(g2, g2p, W3, b2)
    parts3 = _make_agg_kernel(n, g3.shape[0], dtbl, e)(g3p, edge_flat)
    out, h = pl.pallas_call(
        functools.partial(_final_body, n_nodes=n),
        out_shape=[jax.ShapeDtypeStruct((n, Wc.shape[1]), f32),
                   jax.ShapeDtypeStruct((n, g3.shape[0]), f32)],
    )(parts3, g3, dis, b3.reshape(-1, 1), Wc, bc)
    return (out, h)


# speedup vs baseline: 1.0998x; 1.0998x over previous
import functools

import jax
import jax.numpy as jnp
from jax import lax
from jax.experimental import pallas as pl
from jax.experimental.pallas import tpu as pltpu
from jax.experimental.pallas import tpu_sc as plsc

NW = 32
LANES = 16


def _ceil_to(x, m):
    return (x + m - 1) // m * m


def _make_deg_kernel(n_nodes, dtbl, e):
    epw = e // NW
    unroll = 5 if (epw // LANES) % 5 == 0 else 1
    mesh = plsc.VectorSubcoreMesh(core_axis_name="c", subcore_axis_name="s")

    @functools.partial(
        pl.kernel,
        out_type=jax.ShapeDtypeStruct((NW, dtbl), jnp.float32),
        mesh=mesh,
        compiler_params=pltpu.CompilerParams(needs_layout_passes=False),
        scratch_types=[
            pltpu.VMEM((epw,), jnp.int32),
            pltpu.VMEM((dtbl,), jnp.float32),
            pltpu.SemaphoreType.DMA,
        ],
    )
    def deg_kernel(edge_hbm, out_hbm, dst_v, cnt_v, sem):
        cid = lax.axis_index("c")
        sid = lax.axis_index("s")
        wid = sid * 2 + cid
        base = wid * epw
        cp = pltpu.make_async_copy(edge_hbm.at[pl.ds(e + base, epw)], dst_v,
                                   sem)
        cp.start()

        zeros = jnp.zeros((LANES,), jnp.float32)

        def zbody(i, _):
            cnt_v[pl.ds(i * LANES, LANES)] = zeros
            return 0

        lax.fori_loop(0, dtbl // LANES, zbody, 0, unroll=4)
        cp.wait()

        ones = jnp.ones((LANES,), jnp.float32)

        def ebody(i, _):
            for u in range(unroll):
                d16 = dst_v[pl.ds((i * unroll + u) * LANES, LANES)]
                plsc.addupdate_scatter(cnt_v, [d16], ones)
            return 0

        lax.fori_loop(0, epw // LANES // unroll, ebody, 0)
        pltpu.sync_copy(cnt_v, out_hbm.at[wid])

    return deg_kernel


def _make_agg_kernel(n_nodes, d, dtbl, e):
    epw = e // NW
    d2 = d // 2
    unroll = 5 if (epw // LANES) % 5 == 0 else 1
    mesh = plsc.VectorSubcoreMesh(core_axis_name="c", subcore_axis_name="s")

    scratch = [pltpu.VMEM((d2, n_nodes), jnp.int32),
               pltpu.VMEM((d, dtbl), jnp.float32),
               pltpu.VMEM((epw,), jnp.int32),
               pltpu.VMEM((epw,), jnp.int32)] + [pltpu.SemaphoreType.DMA] * 3

    @functools.partial(
        pl.kernel,
        out_type=jax.ShapeDtypeStruct((NW, d, dtbl), jnp.float32),
        mesh=mesh,
        compiler_params=pltpu.CompilerParams(needs_layout_passes=False),
        scratch_types=scratch,
    )
    def agg_kernel(gp_hbm, edge_hbm, out_hbm,
                   g_v, acc_v, src_v, dst_v, s0, s2, s3):
        cid = lax.axis_index("c")
        sid = lax.axis_index("s")
        wid = sid * 2 + cid
        base = wid * epw
        copies = [
            pltpu.make_async_copy(gp_hbm, g_v, s0),
            pltpu.make_async_copy(edge_hbm.at[pl.ds(base, epw)], src_v, s2),
            pltpu.make_async_copy(edge_hbm.at[pl.ds(e + base, epw)], dst_v,
                                  s3),
        ]
        for cp in copies:
            cp.start()

        zeros = jnp.zeros((LANES,), jnp.float32)

        def zbody(i, _):
            for j in range(d):
                acc_v[j, pl.ds(i * LANES, LANES)] = zeros
            return 0

        lax.fori_loop(0, dtbl // LANES, zbody, 0, unroll=2)
        for cp in copies:
            cp.wait()

        rows = [jnp.full((LANES,), k, jnp.int32) for k in range(d2)]
        arows = [jnp.full((LANES,), j, jnp.int32) for j in range(d)]
        himask = jnp.full((LANES,), -65536, jnp.int32)

        def ebody(i, _):
            for u in range(unroll):
                off = (i * unroll + u) * LANES
                s16 = src_v[pl.ds(off, LANES)]
                d16 = dst_v[pl.ds(off, LANES)]
                for k in range(d2):
                    w16 = plsc.load_gather(g_v, [rows[k], s16])
                    lo = plsc.bitcast(w16 << 16, jnp.float32)
                    hi = plsc.bitcast(w16 & himask, jnp.float32)
                    plsc.addupdate_scatter(acc_v, [arows[2 * k], d16], lo)
                    plsc.addupdate_scatter(acc_v, [arows[2 * k + 1], d16],
                                           hi)
            return 0

        lax.fori_loop(0, epw // LANES // unroll, ebody, 0)
        pltpu.sync_copy(acc_v, out_hbm.at[wid])

    return agg_kernel


def _mm_body(x_ref, w1_ref, p1_ref):
    p1_ref[...] = lax.dot_general(
        w1_ref[...], x_ref[...],
        dimension_numbers=(((0,), (1,)), ((), ())),
        preferred_element_type=jnp.float32)


def _write_packed(g, gp_ref):
    u = lax.bitcast_convert_type(g, jnp.int32) + 0x8000
    for k in range(g.shape[0] // 2):
        gp_ref[k, :] = (lax.shift_right_logical(u[2 * k], 16)
                        | (u[2 * k + 1] & (-65536)))


def _dis_body(degp_ref, p1_ref, dis_ref, g1_ref, g1p_ref, *, n_nodes):
    deg = jnp.sum(degp_ref[...], axis=0) + 1.0
    dis = lax.rsqrt(deg)
    dis_ref[...] = dis
    g1 = p1_ref[...] * dis[:n_nodes]
    g1_ref[...] = g1
    _write_packed(g1, g1p_ref)


def _post_body(part_ref, g_ref, dis_ref, b_ref, w_ref, o_ref, op_ref,
               *, n_nodes):
    dis = dis_ref[...][:n_nodes]
    s = jnp.sum(part_ref[...], axis=0)[:, :n_nodes] + g_ref[...]
    h = jnp.tanh(dis * s + b_ref[...])
    g_next = dis * lax.dot_general(
        w_ref[...], h,
        dimension_numbers=(((0,), (0,)), ((), ())),
        preferred_element_type=jnp.float32)
    o_ref[...] = g_next
    _write_packed(g_next, op_ref)


def _final_body(part_ref, g_ref, dis_ref, b_ref, wc_ref, bc_ref,
                out_ref, h_ref, *, n_nodes):
    dis = dis_ref[...][:n_nodes]
    s = jnp.sum(part_ref[...], axis=0)[:, :n_nodes] + g_ref[...]
    h = jnp.tanh(dis * s + b_ref[...])
    h_ref[...] = h
    out_ref[...] = lax.dot_general(
        wc_ref[...], h,
        dimension_numbers=(((0,), (0,)), ((), ())),
        preferred_element_type=jnp.float32) + bc_ref[...]


def kernel(x, edge_index, W1, b1, W2, b2, W3, b3, Wc, bc):
    n = x.shape[0]
    e = edge_index.shape[1]
    f32 = jnp.float32
    dtbl = _ceil_to(n, LANES)

    edge_flat = edge_index.reshape(-1)
    degp = _make_deg_kernel(n, dtbl, e)(edge_flat)

    p1t = pl.pallas_call(
        _mm_body,
        out_shape=jax.ShapeDtypeStruct((W1.shape[1], n), f32),
    )(x, W1)

    dis, g1, g1p = pl.pallas_call(
        functools.partial(_dis_body, n_nodes=n),
        out_shape=[jax.ShapeDtypeStruct((dtbl,), f32),
                   jax.ShapeDtypeStruct((W1.shape[1], n), f32),
                   jax.ShapeDtypeStruct((W1.shape[1] // 2, n), jnp.int32)],
    )(degp, p1t)

    def layer(gt, gtp, w_next, b):
        d = gt.shape[0]
        parts = _make_agg_kernel(n, d, dtbl, e)(gtp, edge_flat)
        nd = w_next.shape[1]
        return pl.pallas_call(
            functools.partial(_post_body, n_nodes=n),
            out_shape=[jax.ShapeDtypeStruct((nd, n), f32),
                       jax.ShapeDtypeStruct((nd // 2, n), jnp.int32)],
        )(parts, gt, dis, b.reshape(-1, 1), w_next)

    g2, g2p = layer(g1, g1p, W2, b1)
    g3, g3p = layer(g2, g2p, W3, b2)
    parts3 = _make_agg_kernel(n, g3.shape[0], dtbl, e)(g3p, edge_flat)
    outt, ht = pl.pallas_call(
        functools.partial(_final_body, n_nodes=n),
        out_shape=[jax.ShapeDtypeStruct((Wc.shape[1], n), f32),
                   jax.ShapeDtypeStruct((g3.shape[0], n), f32)],
    )(parts3, g3, dis, b3.reshape(-1, 1), Wc, bc.reshape(-1, 1))
    return (outt.T, ht.T)


# speedup vs baseline: 1.1408x; 1.0373x over previous
import functools

import jax
import jax.numpy as jnp
from jax import lax
from jax.experimental import pallas as pl
from jax.experimental.pallas import tpu as pltpu
from jax.experimental.pallas import tpu_sc as plsc

NW = 32
LANES = 16


def _ceil_to(x, m):
    return (x + m - 1) // m * m


def _make_deg_kernel(n_nodes, dtbl, e):
    epw = e // NW
    unroll = 5 if (epw // LANES) % 5 == 0 else 1
    mesh = plsc.VectorSubcoreMesh(core_axis_name="c", subcore_axis_name="s")

    @functools.partial(
        pl.kernel,
        out_type=jax.ShapeDtypeStruct((NW, dtbl), jnp.float32),
        mesh=mesh,
        compiler_params=pltpu.CompilerParams(needs_layout_passes=False),
        scratch_types=[
            pltpu.VMEM((epw,), jnp.int32),
            pltpu.VMEM((dtbl,), jnp.float32),
            pltpu.SemaphoreType.DMA,
        ],
    )
    def deg_kernel(edge_hbm, out_hbm, dst_v, cnt_v, sem):
        cid = lax.axis_index("c")
        sid = lax.axis_index("s")
        wid = sid * 2 + cid
        base = wid * epw
        cp = pltpu.make_async_copy(edge_hbm.at[pl.ds(e + base, epw)], dst_v,
                                   sem)
        cp.start()

        zeros = jnp.zeros((LANES,), jnp.float32)

        def zbody(i, _):
            cnt_v[pl.ds(i * LANES, LANES)] = zeros
            return 0

        lax.fori_loop(0, dtbl // LANES, zbody, 0, unroll=4)
        cp.wait()

        ones = jnp.ones((LANES,), jnp.float32)

        def ebody(i, _):
            for u in range(unroll):
                d16 = dst_v[pl.ds((i * unroll + u) * LANES, LANES)]
                plsc.addupdate_scatter(cnt_v, [d16], ones)
            return 0

        lax.fori_loop(0, epw // LANES // unroll, ebody, 0)
        pltpu.sync_copy(cnt_v, out_hbm.at[wid])

    return deg_kernel


def _make_agg_kernel(n_nodes, d, dtbl, e):
    epw = e // NW
    d2 = d // 2
    unroll = 5 if (epw // LANES) % 5 == 0 else 1
    mesh = plsc.VectorSubcoreMesh(core_axis_name="c", subcore_axis_name="s")

    scratch = [pltpu.VMEM((d2, n_nodes), jnp.int32),
               pltpu.VMEM((d, dtbl), jnp.int32),
               pltpu.VMEM((epw,), jnp.int32),
               pltpu.VMEM((epw,), jnp.int32)] + [pltpu.SemaphoreType.DMA] * 3

    @functools.partial(
        pl.kernel,
        out_type=jax.ShapeDtypeStruct((NW, d, dtbl), jnp.int32),
        mesh=mesh,
        compiler_params=pltpu.CompilerParams(needs_layout_passes=False),
        scratch_types=scratch,
    )
    def agg_kernel(gp_hbm, edge_hbm, out_hbm,
                   g_v, acc_v, src_v, dst_v, s0, s2, s3):
        cid = lax.axis_index("c")
        sid = lax.axis_index("s")
        wid = sid * 2 + cid
        base = wid * epw
        copies = [
            pltpu.make_async_copy(gp_hbm, g_v, s0),
            pltpu.make_async_copy(edge_hbm.at[pl.ds(base, epw)], src_v, s2),
            pltpu.make_async_copy(edge_hbm.at[pl.ds(e + base, epw)], dst_v,
                                  s3),
        ]
        for cp in copies:
            cp.start()

        zeros = jnp.zeros((LANES,), jnp.int32)

        def zbody(i, _):
            for j in range(d):
                acc_v[j, pl.ds(i * LANES, LANES)] = zeros
            return 0

        lax.fori_loop(0, dtbl // LANES, zbody, 0, unroll=2)
        for cp in copies:
            cp.wait()

        rows = [jnp.full((LANES,), k, jnp.int32) for k in range(d2)]
        arows = [jnp.full((LANES,), j, jnp.int32) for j in range(d)]

        def ebody(i, _):
            for u in range(unroll):
                off = (i * unroll + u) * LANES
                s16 = src_v[pl.ds(off, LANES)]
                d16 = dst_v[pl.ds(off, LANES)]
                for k in range(d2):
                    w16 = plsc.load_gather(g_v, [rows[k], s16])
                    lo = (w16 << 16) >> 16
                    hi = w16 >> 16
                    plsc.addupdate_scatter(acc_v, [arows[2 * k], d16], lo)
                    plsc.addupdate_scatter(acc_v, [arows[2 * k + 1], d16],
                                           hi)
            return 0

        lax.fori_loop(0, epw // LANES // unroll, ebody, 0)
        pltpu.sync_copy(acc_v, out_hbm.at[wid])

    return agg_kernel


def _mm_body(x_ref, w1_ref, p1_ref):
    p1_ref[...] = lax.dot_general(
        w1_ref[...], x_ref[...],
        dimension_numbers=(((0,), (1,)), ((), ())),
        preferred_element_type=jnp.float32)


SCALE_BOUND = 16.0
_Q = 32767.0 / SCALE_BOUND


def _write_packed(g, gp_ref):
    v = jnp.clip(jnp.rint(g * _Q), -32767.0, 32767.0).astype(jnp.int32)
    for k in range(g.shape[0] // 2):
        gp_ref[k, :] = (v[2 * k] & 0xFFFF) | (v[2 * k + 1] << 16)


def _dis_body(degp_ref, p1_ref, dis_ref, g1_ref, g1p_ref, *, n_nodes):
    deg = jnp.sum(degp_ref[...], axis=0) + 1.0
    dis = lax.rsqrt(deg)
    dis_ref[...] = dis
    g1 = p1_ref[...] * dis[:n_nodes]
    g1_ref[...] = g1
    _write_packed(g1, g1p_ref)


def _post_body(part_ref, g_ref, dis_ref, b_ref, w_ref, o_ref, op_ref,
               *, n_nodes):
    dis = dis_ref[...][:n_nodes]
    si = jnp.sum(part_ref[...], axis=0)[:, :n_nodes]
    s = si.astype(jnp.float32) * (1.0 / _Q) + g_ref[...]
    h = jnp.tanh(dis * s + b_ref[...])
    g_next = dis * lax.dot_general(
        w_ref[...], h,
        dimension_numbers=(((0,), (0,)), ((), ())),
        preferred_element_type=jnp.float32)
    o_ref[...] = g_next
    _write_packed(g_next, op_ref)


def _final_body(part_ref, g_ref, dis_ref, b_ref, wc_ref, bc_ref,
                out_ref, h_ref, *, n_nodes):
    dis = dis_ref[...][:n_nodes]
    si = jnp.sum(part_ref[...], axis=0)[:, :n_nodes]
    s = si.astype(jnp.float32) * (1.0 / _Q) + g_ref[...]
    h = jnp.tanh(dis * s + b_ref[...])
    h_ref[...] = h
    out_ref[...] = lax.dot_general(
        wc_ref[...], h,
        dimension_numbers=(((0,), (0,)), ((), ())),
        preferred_element_type=jnp.float32) + bc_ref[...]


def kernel(x, edge_index, W1, b1, W2, b2, W3, b3, Wc, bc):
    n = x.shape[0]
    e = edge_index.shape[1]
    f32 = jnp.float32
    dtbl = _ceil_to(n, LANES)

    edge_flat = edge_index.reshape(-1)
    degp = _make_deg_kernel(n, dtbl, e)(edge_flat)

    p1t = pl.pallas_call(
        _mm_body,
        out_shape=jax.ShapeDtypeStruct((W1.shape[1], n), f32),
    )(x, W1)

    dis, g1, g1p = pl.pallas_call(
        functools.partial(_dis_body, n_nodes=n),
        out_shape=[jax.ShapeDtypeStruct((dtbl,), f32),
                   jax.ShapeDtypeStruct((W1.shape[1], n), f32),
                   jax.ShapeDtypeStruct((W1.shape[1] // 2, n), jnp.int32)],
    )(degp, p1t)

    def layer(gt, gtp, w_next, b):
        d = gt.shape[0]
        parts = _make_agg_kernel(n, d, dtbl, e)(gtp, edge_flat)
        nd = w_next.shape[1]
        return pl.pallas_call(
            functools.partial(_post_body, n_nodes=n),
            out_shape=[jax.ShapeDtypeStruct((nd, n), f32),
                       jax.ShapeDtypeStruct((nd // 2, n), jnp.int32)],
        )(parts, gt, dis, b.reshape(-1, 1), w_next)

    g2, g2p = layer(g1, g1p, W2, b1)
    g3, g3p = layer(g2, g2p, W3, b2)
    parts3 = _make_agg_kernel(n, g3.shape[0], dtbl, e)(g3p, edge_flat)
    outt, ht = pl.pallas_call(
        functools.partial(_final_body, n_nodes=n),
        out_shape=[jax.ShapeDtypeStruct((Wc.shape[1], n), f32),
                   jax.ShapeDtypeStruct((g3.shape[0], n), f32)],
    )(parts3, g3, dis, b3.reshape(-1, 1), Wc, bc.reshape(-1, 1))
    return (outt.T, ht.T)


# speedup vs baseline: 1.1495x; 1.0076x over previous
import functools

import jax
import jax.numpy as jnp
from jax import lax
from jax.experimental import pallas as pl
from jax.experimental.pallas import tpu as pltpu
from jax.experimental.pallas import tpu_sc as plsc

NW = 32
LANES = 16


def _ceil_to(x, m):
    return (x + m - 1) // m * m


def _make_deg_kernel(n_nodes, dtbl, e):
    epw = e // NW
    unroll = 5 if (epw // LANES) % 5 == 0 else 1
    mesh = plsc.VectorSubcoreMesh(core_axis_name="c", subcore_axis_name="s")

    @functools.partial(
        pl.kernel,
        out_type=jax.ShapeDtypeStruct((NW, dtbl), jnp.float32),
        mesh=mesh,
        compiler_params=pltpu.CompilerParams(needs_layout_passes=False),
        scratch_types=[
            pltpu.VMEM((epw,), jnp.int32),
            pltpu.VMEM((dtbl,), jnp.float32),
            pltpu.SemaphoreType.DMA,
        ],
    )
    def deg_kernel(edge_hbm, out_hbm, dst_v, cnt_v, sem):
        cid = lax.axis_index("c")
        sid = lax.axis_index("s")
        wid = sid * 2 + cid
        base = wid * epw
        cp = pltpu.make_async_copy(edge_hbm.at[pl.ds(e + base, epw)], dst_v,
                                   sem)
        cp.start()

        zeros = jnp.zeros((LANES,), jnp.float32)

        def zbody(i, _):
            cnt_v[pl.ds(i * LANES, LANES)] = zeros
            return 0

        lax.fori_loop(0, dtbl // LANES, zbody, 0, unroll=4)
        cp.wait()

        ones = jnp.ones((LANES,), jnp.float32)

        def ebody(i, _):
            for u in range(unroll):
                d16 = dst_v[pl.ds((i * unroll + u) * LANES, LANES)]
                plsc.addupdate_scatter(cnt_v, [d16], ones)
            return 0

        lax.fori_loop(0, epw // LANES // unroll, ebody, 0)
        pltpu.sync_copy(cnt_v, out_hbm.at[wid])

    return deg_kernel


def _make_agg_kernel(n_nodes, d, dtbl, e):
    epw = e // NW
    d2 = d // 2
    unroll = 5 if (epw // LANES) % 5 == 0 else 1
    mesh = plsc.VectorSubcoreMesh(core_axis_name="c", subcore_axis_name="s")

    scratch = [pltpu.VMEM((d2, n_nodes), jnp.int32),
               pltpu.VMEM((d, dtbl), jnp.int32),
               pltpu.VMEM((epw,), jnp.int32),
               pltpu.VMEM((epw,), jnp.int32)] + [pltpu.SemaphoreType.DMA] * 3

    @functools.partial(
        pl.kernel,
        out_type=jax.ShapeDtypeStruct((NW, d, dtbl), jnp.int32),
        mesh=mesh,
        compiler_params=pltpu.CompilerParams(needs_layout_passes=False),
        scratch_types=scratch,
    )
    def agg_kernel(gp_hbm, edge_hbm, out_hbm,
                   g_v, acc_v, src_v, dst_v, s0, s2, s3):
        cid = lax.axis_index("c")
        sid = lax.axis_index("s")
        wid = sid * 2 + cid
        base = wid * epw
        copies = [
            pltpu.make_async_copy(gp_hbm, g_v, s0),
            pltpu.make_async_copy(edge_hbm.at[pl.ds(base, epw)], src_v, s2),
            pltpu.make_async_copy(edge_hbm.at[pl.ds(e + base, epw)], dst_v,
                                  s3),
        ]
        for cp in copies:
            cp.start()

        zeros = jnp.zeros((LANES,), jnp.int32)

        def zbody(i, _):
            for j in range(d):
                acc_v[j, pl.ds(i * LANES, LANES)] = zeros
            return 0

        lax.fori_loop(0, dtbl // LANES, zbody, 0, unroll=2)
        for cp in copies:
            cp.wait()

        rows = [jnp.full((LANES,), k, jnp.int32) for k in range(d2)]
        arows = [jnp.full((LANES,), j, jnp.int32) for j in range(d)]

        def ebody(i, _):
            for u in range(unroll):
                off = (i * unroll + u) * LANES
                s16 = src_v[pl.ds(off, LANES)]
                d16 = dst_v[pl.ds(off, LANES)]
                for k in range(d2):
                    w16 = plsc.load_gather(g_v, [rows[k], s16])
                    lo = w16 & 0xFFFF
                    hi = lax.shift_right_logical(w16, 16)
                    plsc.addupdate_scatter(acc_v, [arows[2 * k], d16], lo)
                    plsc.addupdate_scatter(acc_v, [arows[2 * k + 1], d16],
                                           hi)
            return 0

        lax.fori_loop(0, epw // LANES // unroll, ebody, 0)
        pltpu.sync_copy(acc_v, out_hbm.at[wid])

    return agg_kernel


def _mm_body(x_ref, w1_ref, p1_ref):
    p1_ref[...] = lax.dot_general(
        w1_ref[...], x_ref[...],
        dimension_numbers=(((0,), (1,)), ((), ())),
        preferred_element_type=jnp.float32)


SCALE_BOUND = 16.0
_Q = 32767.0 / SCALE_BOUND


def _write_packed(g, gp_ref):
    v = (jnp.clip(jnp.rint(g * _Q), -32767.0, 32767.0).astype(jnp.int32)
         + 32768)
    for k in range(g.shape[0] // 2):
        gp_ref[k, :] = v[2 * k] | (v[2 * k + 1] << 16)


def _dis_body(degp_ref, p1_ref, dis_ref, g1_ref, g1p_ref, *, n_nodes):
    deg = jnp.sum(degp_ref[...], axis=0) + 1.0
    dis = lax.rsqrt(deg)
    dis_ref[...] = dis
    g1 = p1_ref[...] * dis[:n_nodes]
    g1_ref[...] = g1
    _write_packed(g1, g1p_ref)


def _post_body(part_ref, g_ref, dis_ref, b_ref, w_ref, o_ref, op_ref,
               *, n_nodes):
    dis = dis_ref[...][:n_nodes]
    corr = (jnp.rint(1.0 / (dis * dis)) - 1.0) * 32768.0
    si = jnp.sum(part_ref[...], axis=0)[:, :n_nodes]
    s = (si.astype(jnp.float32) - corr) * (1.0 / _Q) + g_ref[...]
    h = jnp.tanh(dis * s + b_ref[...])
    g_next = dis * lax.dot_general(
        w_ref[...], h,
        dimension_numbers=(((0,), (0,)), ((), ())),
        preferred_element_type=jnp.float32)
    o_ref[...] = g_next
    _write_packed(g_next, op_ref)


def _final_body(part_ref, g_ref, dis_ref, b_ref, wc_ref, bc_ref,
                out_ref, h_ref, *, n_nodes):
    dis = dis_ref[...][:n_nodes]
    corr = (jnp.rint(1.0 / (dis * dis)) - 1.0) * 32768.0
    si = jnp.sum(part_ref[...], axis=0)[:, :n_nodes]
    s = (si.astype(jnp.float32) - corr) * (1.0 / _Q) + g_ref[...]
    h = jnp.tanh(dis * s + b_ref[...])
    h_ref[...] = h
    out_ref[...] = lax.dot_general(
        wc_ref[...], h,
        dimension_numbers=(((0,), (0,)), ((), ())),
        preferred_element_type=jnp.float32) + bc_ref[...]


def kernel(x, edge_index, W1, b1, W2, b2, W3, b3, Wc, bc):
    n = x.shape[0]
    e = edge_index.shape[1]
    f32 = jnp.float32
    dtbl = _ceil_to(n, LANES)

    edge_flat = edge_index.reshape(-1)
    degp = _make_deg_kernel(n, dtbl, e)(edge_flat)

    p1t = pl.pallas_call(
        _mm_body,
        out_shape=jax.ShapeDtypeStruct((W1.shape[1], n), f32),
    )(x, W1)

    dis, g1, g1p = pl.pallas_call(
        functools.partial(_dis_body, n_nodes=n),
        out_shape=[jax.ShapeDtypeStruct((dtbl,), f32),
                   jax.ShapeDtypeStruct((W1.shape[1], n), f32),
                   jax.ShapeDtypeStruct((W1.shape[1] // 2, n), jnp.int32)],
    )(degp, p1t)

    def layer(gt, gtp, w_next, b):
        d = gt.shape[0]
        parts = _make_agg_kernel(n, d, dtbl, e)(gtp, edge_flat)
        nd = w_next.shape[1]
        return pl.pallas_call(
            functools.partial(_post_body, n_nodes=n),
            out_shape=[jax.ShapeDtypeStruct((nd, n), f32),
                       jax.ShapeDtypeStruct((nd // 2, n), jnp.int32)],
        )(parts, gt, dis, b.reshape(-1, 1), w_next)

    g2, g2p = layer(g1, g1p, W2, b1)
    g3, g3p = layer(g2, g2p, W3, b2)
    parts3 = _make_agg_kernel(n, g3.shape[0], dtbl, e)(g3p, edge_flat)
    outt, ht = pl.pallas_call(
        functools.partial(_final_body, n_nodes=n),
        out_shape=[jax.ShapeDtypeStruct((Wc.shape[1], n), f32),
                   jax.ShapeDtypeStruct((g3.shape[0], n), f32)],
    )(parts3, g3, dis, b3.reshape(-1, 1), Wc, bc.reshape(-1, 1))
    return (outt.T, ht.T)
